# baseline (speedup 1.0000x reference)
import functools

import jax
import jax.numpy as jnp
from jax import lax
from jax.experimental import pallas as pl
from jax.experimental.pallas import tpu as pltpu
from jax.experimental.pallas import tpu_sc as plsc

N = 10000
E = 320000
NC = 2
NS = 16
CH = 112
NBUF = 3
NCHUNK = 180
PH = 15
PC = NCHUNK // PH
EPAD = NS * NCHUNK * CH
ACC_ROWS = 10016
ZROWS = 626
DUMMY = N
WB = 624
WB_TAIL = N - NS * WB
HB = 1920
HGRID = EPAD // HB


def _pack_edges(edge):
    dst = edge[0].astype(jnp.int32)
    src = edge[1].astype(jnp.int32)
    pad = EPAD - E
    dst = jnp.concatenate([dst, jnp.full((pad,), DUMMY, jnp.int32)])
    src = jnp.concatenate([src, jnp.zeros((pad,), jnp.int32)])
    return (src.reshape(NS, PH, PC, CH),
            dst.reshape(NS, PH, PC, CH))


def _make_agg():
    mesh = plsc.VectorSubcoreMesh(core_axis_name="c", subcore_axis_name="s")

    @functools.partial(
        pl.kernel,
        mesh=mesh,
        out_type=(jax.ShapeDtypeStruct((N, 128), jnp.float32),
                  jax.ShapeDtypeStruct((N, 128), jnp.float32)),
        scratch_types=[
            pltpu.VMEM((PC, CH), jnp.int32),
            pltpu.VMEM((PC, CH), jnp.int32),
            pltpu.VMEM((NBUF, CH, 128), jnp.float32),
            pltpu.VMEM_SHARED((ACC_ROWS, 128), jnp.float32),
        ] + [pltpu.SemaphoreType.DMA] * (2 * NBUF),
    )
    def agg(ta_hbm, tp_hbm, srca_hbm, dsta_hbm, srcp_hbm, dstp_hbm,
            zeros_hbm, outa_hbm, outp_hbm, src_v, dst_v, rows_v, acc_s,
            *sems):
        gsem = sems[:NBUF]
        ssem = sems[NBUF:]
        cid = lax.axis_index("c")
        sid = lax.axis_index("s")

        def pipe(table_hbm, src_hbm, dst_hbm, out_hbm):
            pltpu.sync_copy(zeros_hbm,
                            acc_s.at[pl.ds(sid * ZROWS, ZROWS)])
            plsc.subcore_barrier()

            def gather(j, b):
                pltpu.async_copy(
                    table_hbm.at[src_v.at[j]], rows_v.at[b], gsem[b])

            def gather_wait(j, b):
                pltpu.make_async_copy(
                    table_hbm.at[src_v.at[j]], rows_v.at[b], gsem[b]).wait()

            def scatter(j, b):
                pltpu.async_copy(
                    rows_v.at[b], acc_s.at[dst_v.at[j]], ssem[b], add=True)

            def scatter_wait(j, b):
                pltpu.make_async_copy(
                    rows_v.at[b], acc_s.at[dst_v.at[j]], ssem[b]).wait()

            def phase(p, carry):
                pltpu.sync_copy(src_hbm.at[sid, p], src_v)
                pltpu.sync_copy(dst_hbm.at[sid, p], dst_v)
                for b in range(NBUF):
                    gather(b, b)

                def outer(i, c):
                    g = i * NBUF
                    for b in range(NBUF):
                        gather_wait(g + b, b)
                        scatter(g + b, b)
                    for b in range(NBUF):
                        scatter_wait(g + b, b)
                        gather(g + NBUF + b, b)
                    return c

                lax.fori_loop(0, PC // NBUF - 1, outer, 0)
                g = PC - NBUF
                for b in range(NBUF):
                    gather_wait(g + b, b)
                    scatter(g + b, b)
                for b in range(NBUF):
                    scatter_wait(g + b, b)
                return carry

            lax.fori_loop(0, PH, phase, 0)
            plsc.subcore_barrier()
            pltpu.sync_copy(acc_s.at[pl.ds(sid * WB, WB)],
                            out_hbm.at[pl.ds(sid * WB, WB)])

            @pl.when(sid == 0)
            def _():
                pltpu.sync_copy(acc_s.at[pl.ds(NS * WB, WB_TAIL)],
                                out_hbm.at[pl.ds(NS * WB, WB_TAIL)])

        @pl.when(cid == 0)
        def _():
            pipe(ta_hbm, srca_hbm, dsta_hbm, outa_hbm)

        @pl.when(cid == 1)
        def _():
            pipe(tp_hbm, srcp_hbm, dstp_hbm, outp_hbm)

    return agg


def _tc_count(dst_l):
    def body(l_ref, o_ref):
        @pl.when(pl.program_id(0) == 0)
        def _():
            o_ref[...] = jnp.zeros_like(o_ref)

        q = l_ref[0] >> 7
        r = l_ref[0] & 127
        oh_q = (lax.broadcasted_iota(jnp.int32, (80, HB), 0)
                == jnp.broadcast_to(q, (80, HB))).astype(jnp.bfloat16)
        oh_r = (lax.broadcasted_iota(jnp.int32, (128, HB), 0)
                == jnp.broadcast_to(r, (128, HB))).astype(jnp.bfloat16)
        o_ref[...] += lax.dot_general(
            oh_q, oh_r, (((1,), (1,)), ((), ())),
            preferred_element_type=jnp.float32)

    return pl.pallas_call(
        body,
        grid=(HGRID,),
        in_specs=[pl.BlockSpec((1, 1, HB), lambda i: (i, 0, 0))],
        out_specs=pl.BlockSpec((80, 128), lambda i: (0, 0)),
        out_shape=jax.ShapeDtypeStruct((80, 128), jnp.float32),
    )(dst_l)


def _tc_layer1(summ, cnt, w0):
    blk = 1000

    def body(p_ref, c_ref, w_ref, emb_ref, dinv_ref):
        d = 1.0 / jnp.maximum(c_ref[...], 1.0)
        mn = p_ref[...] * d
        emb_ref[...] = jnp.maximum(
            jnp.dot(mn, w_ref[...], preferred_element_type=jnp.float32), 0.0)
        dinv_ref[...] = d

    return pl.pallas_call(
        body,
        grid=(N // blk,),
        in_specs=[
            pl.BlockSpec((blk, 128), lambda i: (i, 0)),
            pl.BlockSpec((blk, 1), lambda i: (i, 0)),
            pl.BlockSpec((128, 128), lambda i: (0, 0)),
        ],
        out_specs=[
            pl.BlockSpec((blk, 128), lambda i: (i, 0)),
            pl.BlockSpec((blk, 1), lambda i: (i, 0)),
        ],
        out_shape=[
            jax.ShapeDtypeStruct((N, 128), jnp.float32),
            jax.ShapeDtypeStruct((N, 1), jnp.float32),
        ],
    )(summ, cnt, w0)


def _tc_layer2(qsum, dinv, ft, w1, wv, wf, b):
    blk = 1000

    def body(q_ref, d_ref, f_ref, w1_ref, wv_ref, wf_ref, b_ref, o_ref):
        x = q_ref[...] * d_ref[...]
        v = jnp.maximum(
            jnp.dot(x, w1_ref[...], preferred_element_type=jnp.float32), 0.0)
        o_ref[...] = (
            jnp.dot(v, wv_ref[...], preferred_element_type=jnp.float32)
            + jnp.dot(f_ref[...], wf_ref[...],
                      preferred_element_type=jnp.float32)
            + b_ref[...])

    return pl.pallas_call(
        body,
        grid=(N // blk,),
        in_specs=[
            pl.BlockSpec((blk, 128), lambda i: (i, 0)),
            pl.BlockSpec((blk, 1), lambda i: (i, 0)),
            pl.BlockSpec((blk, 128), lambda i: (i, 0)),
            pl.BlockSpec((128, 128), lambda i: (0, 0)),
            pl.BlockSpec((128, 128), lambda i: (0, 0)),
            pl.BlockSpec((128, 128), lambda i: (0, 0)),
            pl.BlockSpec((1, 128), lambda i: (0, 0)),
        ],
        out_specs=pl.BlockSpec((blk, 128), lambda i: (i, 0)),
        out_shape=jax.ShapeDtypeStruct((N, 128), jnp.float32),
    )(qsum, dinv, ft, w1, wv, wf, b)


def _cnt_col(hist):
    return hist.reshape(80 * 128)[:N].reshape(N, 1)


def kernel(ft_a, ft_p, edge_a2p, edge_p2a, W0_ap, W0_pa, W1_ap, W1_pa,
           Wfc_a, bfc_a, Wfc_p, bfc_p):
    src_a, dst_a = _pack_edges(edge_a2p)
    src_p, dst_p = _pack_edges(edge_p2a)
    zeros = jnp.zeros((ZROWS, 128), jnp.float32)
    agg = _make_agg()

    cnt_a = _cnt_col(_tc_count(dst_a.reshape(HGRID, 1, HB)))
    cnt_p = _cnt_col(_tc_count(dst_p.reshape(HGRID, 1, HB)))

    sum_a1, sum_p1 = agg(ft_p, ft_a, src_a, dst_a, src_p, dst_p, zeros)
    emb1_a, dinv_a = _tc_layer1(sum_a1, cnt_a, W0_ap)
    emb1_p, dinv_p = _tc_layer1(sum_p1, cnt_p, W0_pa)

    sum_a2, sum_p2 = agg(emb1_p, emb1_a, src_a, dst_a, src_p, dst_p, zeros)
    out_a = _tc_layer2(sum_a2, dinv_a, ft_a, W1_ap,
                       Wfc_a[:128], Wfc_a[128:], bfc_a.reshape(1, 128))
    out_p = _tc_layer2(sum_p2, dinv_p, ft_p, W1_pa,
                       Wfc_p[:128], Wfc_p[128:], bfc_p.reshape(1, 128))
    return jnp.concatenate([out_a, out_p], axis=0)


# speedup vs baseline: 1.0020x; 1.0020x over previous
import functools

import jax
import jax.numpy as jnp
from jax import lax
from jax.experimental import pallas as pl
from jax.experimental.pallas import tpu as pltpu
from jax.experimental.pallas import tpu_sc as plsc

N = 10000
E = 320000
NC = 2
NS = 16
CH = 112
NBUF = 3
NCHUNK = 180
PH = 10
PC = NCHUNK // PH
EPAD = NS * NCHUNK * CH
ACC_ROWS = 10016
ZROWS = 626
DUMMY = N
WB = 624
WB_TAIL = N - NS * WB
HB = 1920
HGRID = EPAD // HB


def _pack_edges(edge):
    dst = edge[0].astype(jnp.int32)
    src = edge[1].astype(jnp.int32)
    pad = EPAD - E
    dst = jnp.concatenate([dst, jnp.full((pad,), DUMMY, jnp.int32)])
    src = jnp.concatenate([src, jnp.zeros((pad,), jnp.int32)])
    return jnp.stack([src.reshape(NS, PH, PC, CH),
                      dst.reshape(NS, PH, PC, CH)], axis=2)


def _make_agg():
    mesh = plsc.VectorSubcoreMesh(core_axis_name="c", subcore_axis_name="s")

    @functools.partial(
        pl.kernel,
        mesh=mesh,
        out_type=(jax.ShapeDtypeStruct((N, 128), jnp.float32),
                  jax.ShapeDtypeStruct((N, 128), jnp.float32)),
        scratch_types=[
            pltpu.VMEM((2, PC, CH), jnp.int32),
            pltpu.VMEM((NBUF, CH, 128), jnp.float32),
            pltpu.VMEM_SHARED((ACC_ROWS, 128), jnp.float32),
        ] + [pltpu.SemaphoreType.DMA] * (2 * NBUF),
    )
    def agg(ta_hbm, tp_hbm, edga_hbm, edgp_hbm, zeros_hbm,
            outa_hbm, outp_hbm, idx_v, rows_v, acc_s, *sems):
        gsem = sems[:NBUF]
        ssem = sems[NBUF:]
        cid = lax.axis_index("c")
        sid = lax.axis_index("s")

        def pipe(table_hbm, edg_hbm, out_hbm):
            src_v = idx_v.at[0]
            dst_v = idx_v.at[1]
            pltpu.sync_copy(zeros_hbm,
                            acc_s.at[pl.ds(sid * ZROWS, ZROWS)])
            plsc.subcore_barrier()

            def gather(j, b):
                pltpu.async_copy(
                    table_hbm.at[src_v.at[j]], rows_v.at[b], gsem[b])

            def gather_wait(j, b):
                pltpu.make_async_copy(
                    table_hbm.at[src_v.at[j]], rows_v.at[b], gsem[b]).wait()

            def scatter(j, b):
                pltpu.async_copy(
                    rows_v.at[b], acc_s.at[dst_v.at[j]], ssem[b], add=True)

            def scatter_wait(j, b):
                pltpu.make_async_copy(
                    rows_v.at[b], acc_s.at[dst_v.at[j]], ssem[b]).wait()

            def phase(p, carry):
                pltpu.sync_copy(edg_hbm.at[sid, p], idx_v)
                for b in range(NBUF):
                    gather(b, b)

                def outer(i, c):
                    g = i * NBUF
                    for b in range(NBUF):
                        gather_wait(g + b, b)
                        scatter(g + b, b)
                    for b in range(NBUF):
                        scatter_wait(g + b, b)
                        gather(g + NBUF + b, b)
                    return c

                lax.fori_loop(0, PC // NBUF - 1, outer, 0)
                g = PC - NBUF
                for b in range(NBUF):
                    gather_wait(g + b, b)
                    scatter(g + b, b)
                for b in range(NBUF):
                    scatter_wait(g + b, b)
                return carry

            lax.fori_loop(0, PH, phase, 0)
            plsc.subcore_barrier()
            pltpu.sync_copy(acc_s.at[pl.ds(sid * WB, WB)],
                            out_hbm.at[pl.ds(sid * WB, WB)])

            @pl.when(sid == 0)
            def _():
                pltpu.sync_copy(acc_s.at[pl.ds(NS * WB, WB_TAIL)],
                                out_hbm.at[pl.ds(NS * WB, WB_TAIL)])

        @pl.when(cid == 0)
        def _():
            pipe(ta_hbm, edga_hbm, outa_hbm)

        @pl.when(cid == 1)
        def _():
            pipe(tp_hbm, edgp_hbm, outp_hbm)

    return agg


def _tc_count(dst_l):
    def body(l_ref, o_ref):
        @pl.when(pl.program_id(0) == 0)
        def _():
            o_ref[...] = jnp.zeros_like(o_ref)

        q = l_ref[0] >> 7
        r = l_ref[0] & 127
        oh_q = (lax.broadcasted_iota(jnp.int32, (80, HB), 0)
                == jnp.broadcast_to(q, (80, HB))).astype(jnp.bfloat16)
        oh_r = (lax.broadcasted_iota(jnp.int32, (128, HB), 0)
                == jnp.broadcast_to(r, (128, HB))).astype(jnp.bfloat16)
        o_ref[...] += lax.dot_general(
            oh_q, oh_r, (((1,), (1,)), ((), ())),
            preferred_element_type=jnp.float32)

    return pl.pallas_call(
        body,
        grid=(HGRID,),
        in_specs=[pl.BlockSpec((1, 1, HB), lambda i: (i, 0, 0))],
        out_specs=pl.BlockSpec((80, 128), lambda i: (0, 0)),
        out_shape=jax.ShapeDtypeStruct((80, 128), jnp.float32),
    )(dst_l)


def _tc_layer1(summ, cnt, w0):
    blk = 1000

    def body(p_ref, c_ref, w_ref, emb_ref, dinv_ref):
        d = 1.0 / jnp.maximum(c_ref[...], 1.0)
        mn = p_ref[...] * d
        emb_ref[...] = jnp.maximum(
            jnp.dot(mn, w_ref[...], preferred_element_type=jnp.float32), 0.0)
        dinv_ref[...] = d

    return pl.pallas_call(
        body,
        grid=(N // blk,),
        in_specs=[
            pl.BlockSpec((blk, 128), lambda i: (i, 0)),
            pl.BlockSpec((blk, 1), lambda i: (i, 0)),
            pl.BlockSpec((128, 128), lambda i: (0, 0)),
        ],
        out_specs=[
            pl.BlockSpec((blk, 128), lambda i: (i, 0)),
            pl.BlockSpec((blk, 1), lambda i: (i, 0)),
        ],
        out_shape=[
            jax.ShapeDtypeStruct((N, 128), jnp.float32),
            jax.ShapeDtypeStruct((N, 1), jnp.float32),
        ],
    )(summ, cnt, w0)


def _tc_layer2(qsum, dinv, ft, w1, wv, wf, b):
    blk = 1000

    def body(q_ref, d_ref, f_ref, w1_ref, wv_ref, wf_ref, b_ref, o_ref):
        x = q_ref[...] * d_ref[...]
        v = jnp.maximum(
            jnp.dot(x, w1_ref[...], preferred_element_type=jnp.float32), 0.0)
        o_ref[...] = (
            jnp.dot(v, wv_ref[...], preferred_element_type=jnp.float32)
            + jnp.dot(f_ref[...], wf_ref[...],
                      preferred_element_type=jnp.float32)
            + b_ref[...])

    return pl.pallas_call(
        body,
        grid=(N // blk,),
        in_specs=[
            pl.BlockSpec((blk, 128), lambda i: (i, 0)),
            pl.BlockSpec((blk, 1), lambda i: (i, 0)),
            pl.BlockSpec((blk, 128), lambda i: (i, 0)),
            pl.BlockSpec((128, 128), lambda i: (0, 0)),
            pl.BlockSpec((128, 128), lambda i: (0, 0)),
            pl.BlockSpec((128, 128), lambda i: (0, 0)),
            pl.BlockSpec((1, 128), lambda i: (0, 0)),
        ],
        out_specs=pl.BlockSpec((blk, 128), lambda i: (i, 0)),
        out_shape=jax.ShapeDtypeStruct((N, 128), jnp.float32),
    )(qsum, dinv, ft, w1, wv, wf, b)


def _cnt_col(hist):
    return hist.reshape(80 * 128)[:N].reshape(N, 1)


def kernel(ft_a, ft_p, edge_a2p, edge_p2a, W0_ap, W0_pa, W1_ap, W1_pa,
           Wfc_a, bfc_a, Wfc_p, bfc_p):
    edg_a = _pack_edges(edge_a2p)
    edg_p = _pack_edges(edge_p2a)
    dst_a = edg_a[:, :, 1]
    dst_p = edg_p[:, :, 1]
    zeros = jnp.zeros((ZROWS, 128), jnp.float32)
    agg = _make_agg()

    cnt_a = _cnt_col(_tc_count(dst_a.reshape(HGRID, 1, HB)))
    cnt_p = _cnt_col(_tc_count(dst_p.reshape(HGRID, 1, HB)))

    sum_a1, sum_p1 = agg(ft_p, ft_a, edg_a, edg_p, zeros)
    emb1_a, dinv_a = _tc_layer1(sum_a1, cnt_a, W0_ap)
    emb1_p, dinv_p = _tc_layer1(sum_p1, cnt_p, W0_pa)

    sum_a2, sum_p2 = agg(emb1_p, emb1_a, edg_a, edg_p, zeros)
    out_a = _tc_layer2(sum_a2, dinv_a, ft_a, W1_ap,
                       Wfc_a[:128], Wfc_a[128:], bfc_a.reshape(1, 128))
    out_p = _tc_layer2(sum_p2, dinv_p, ft_p, W1_pa,
                       Wfc_p[:128], Wfc_p[128:], bfc_p.reshape(1, 128))
    return jnp.concatenate([out_a, out_p], axis=0)


# speedup vs baseline: 1.0236x; 1.0216x over previous
import functools

import jax
import jax.numpy as jnp
from jax import lax
from jax.experimental import pallas as pl
from jax.experimental.pallas import tpu as pltpu
from jax.experimental.pallas import tpu_sc as plsc

N = 10000
E = 320000
NC = 2
NS = 16
CH = 112
NBUF = 3
NCHUNK = 180
PH = 10
PC = NCHUNK // PH
EPAD = NS * NCHUNK * CH
ACC_ROWS = 10016
ZROWS = 626
DUMMY = N
WB = 624
WB_TAIL = N - NS * WB
HB = 1920
HGRID = EPAD // HB


def _pack_edges(edge):
    dst = edge[0].astype(jnp.int32)
    src = edge[1].astype(jnp.int32)
    pad = EPAD - E
    dst = jnp.concatenate([dst, jnp.full((pad,), DUMMY, jnp.int32)])
    src = jnp.concatenate([src, jnp.zeros((pad,), jnp.int32)])
    return (src.reshape(NS, PH, PC, CH),
            dst.reshape(NS, PH, PC, CH))


def _make_agg():
    mesh = plsc.VectorSubcoreMesh(core_axis_name="c", subcore_axis_name="s")

    @functools.partial(
        pl.kernel,
        mesh=mesh,
        out_type=(jax.ShapeDtypeStruct((N, 128), jnp.float32),
                  jax.ShapeDtypeStruct((N, 128), jnp.float32)),
        scratch_types=[
            pltpu.VMEM((PC, CH), jnp.int32),
            pltpu.VMEM((PC, CH), jnp.int32),
            pltpu.VMEM((NBUF, CH, 128), jnp.float32),
            pltpu.VMEM_SHARED((ACC_ROWS, 128), jnp.float32),
        ] + [pltpu.SemaphoreType.DMA] * (2 * NBUF),
    )
    def agg(ta_hbm, tp_hbm, srca_hbm, dsta_hbm, srcp_hbm, dstp_hbm,
            zeros_hbm, outa_hbm, outp_hbm, src_v, dst_v, rows_v, acc_s,
            *sems):
        gsem = sems[:NBUF]
        ssem = sems[NBUF:]
        cid = lax.axis_index("c")
        sid = lax.axis_index("s")

        def pipe(table_hbm, src_hbm, dst_hbm, out_hbm):
            pltpu.sync_copy(zeros_hbm,
                            acc_s.at[pl.ds(sid * ZROWS, ZROWS)])
            plsc.subcore_barrier()

            def gather(j, b):
                pltpu.async_copy(
                    table_hbm.at[src_v.at[j]], rows_v.at[b], gsem[b])

            def gather_wait(j, b):
                pltpu.make_async_copy(
                    table_hbm.at[src_v.at[j]], rows_v.at[b], gsem[b]).wait()

            def scatter(j, b):
                pltpu.async_copy(
                    rows_v.at[b], acc_s.at[dst_v.at[j]], ssem[b], add=True)

            def scatter_wait(j, b):
                pltpu.make_async_copy(
                    rows_v.at[b], acc_s.at[dst_v.at[j]], ssem[b]).wait()

            def phase(p, carry):
                pltpu.sync_copy(src_hbm.at[sid, p], src_v)
                pltpu.sync_copy(dst_hbm.at[sid, p], dst_v)
                for b in range(NBUF):
                    gather(b, b)

                def outer(i, c):
                    g = i * NBUF
                    for b in range(NBUF):
                        gather_wait(g + b, b)
                        scatter(g + b, b)
                    for b in range(NBUF):
                        scatter_wait(g + b, b)
                        gather(g + NBUF + b, b)
                    return c

                lax.fori_loop(0, PC // NBUF - 1, outer, 0)
                g = PC - NBUF
                for b in range(NBUF):
                    gather_wait(g + b, b)
                    scatter(g + b, b)
                for b in range(NBUF):
                    scatter_wait(g + b, b)
                return carry

            lax.fori_loop(0, PH, phase, 0)
            plsc.subcore_barrier()
            pltpu.sync_copy(acc_s.at[pl.ds(sid * WB, WB)],
                            out_hbm.at[pl.ds(sid * WB, WB)])

            @pl.when(sid == 0)
            def _():
                pltpu.sync_copy(acc_s.at[pl.ds(NS * WB, WB_TAIL)],
                                out_hbm.at[pl.ds(NS * WB, WB_TAIL)])

        @pl.when(cid == 0)
        def _():
            pipe(ta_hbm, srca_hbm, dsta_hbm, outa_hbm)

        @pl.when(cid == 1)
        def _():
            pipe(tp_hbm, srcp_hbm, dstp_hbm, outp_hbm)

    return agg


def _tc_count(dst_l):
    def body(l_ref, o_ref):
        @pl.when(pl.program_id(0) == 0)
        def _():
            o_ref[...] = jnp.zeros_like(o_ref)

        q = l_ref[0] >> 7
        r = l_ref[0] & 127
        oh_q = (lax.broadcasted_iota(jnp.int32, (80, HB), 0)
                == jnp.broadcast_to(q, (80, HB))).astype(jnp.bfloat16)
        oh_r = (lax.broadcasted_iota(jnp.int32, (128, HB), 0)
                == jnp.broadcast_to(r, (128, HB))).astype(jnp.bfloat16)
        o_ref[...] += lax.dot_general(
            oh_q, oh_r, (((1,), (1,)), ((), ())),
            preferred_element_type=jnp.float32)

    return pl.pallas_call(
        body,
        grid=(HGRID,),
        in_specs=[pl.BlockSpec((1, 1, HB), lambda i: (i, 0, 0))],
        out_specs=pl.BlockSpec((80, 128), lambda i: (0, 0)),
        out_shape=jax.ShapeDtypeStruct((80, 128), jnp.float32),
    )(dst_l)


def _tc_layer1(summ, cnt, w0):
    blk = 1000

    def body(p_ref, c_ref, w_ref, emb_ref, dinv_ref):
        d = 1.0 / jnp.maximum(c_ref[...], 1.0)
        mn = p_ref[...] * d
        emb_ref[...] = jnp.maximum(
            jnp.dot(mn, w_ref[...], preferred_element_type=jnp.float32), 0.0)
        dinv_ref[...] = d

    return pl.pallas_call(
        body,
        grid=(N // blk,),
        in_specs=[
            pl.BlockSpec((blk, 128), lambda i: (i, 0)),
            pl.BlockSpec((blk, 1), lambda i: (i, 0)),
            pl.BlockSpec((128, 128), lambda i: (0, 0)),
        ],
        out_specs=[
            pl.BlockSpec((blk, 128), lambda i: (i, 0)),
            pl.BlockSpec((blk, 1), lambda i: (i, 0)),
        ],
        out_shape=[
            jax.ShapeDtypeStruct((N, 128), jnp.float32),
            jax.ShapeDtypeStruct((N, 1), jnp.float32),
        ],
    )(summ, cnt, w0)


def _tc_layer2(qsum, dinv, ft, w1, wv, wf, b):
    blk = 1000

    def body(q_ref, d_ref, f_ref, w1_ref, wv_ref, wf_ref, b_ref, o_ref):
        x = q_ref[...] * d_ref[...]
        v = jnp.maximum(
            jnp.dot(x, w1_ref[...], preferred_element_type=jnp.float32), 0.0)
        o_ref[...] = (
            jnp.dot(v, wv_ref[...], preferred_element_type=jnp.float32)
            + jnp.dot(f_ref[...], wf_ref[...],
                      preferred_element_type=jnp.float32)
            + b_ref[...])

    return pl.pallas_call(
        body,
        grid=(N // blk,),
        in_specs=[
            pl.BlockSpec((blk, 128), lambda i: (i, 0)),
            pl.BlockSpec((blk, 1), lambda i: (i, 0)),
            pl.BlockSpec((blk, 128), lambda i: (i, 0)),
            pl.BlockSpec((128, 128), lambda i: (0, 0)),
            pl.BlockSpec((128, 128), lambda i: (0, 0)),
            pl.BlockSpec((128, 128), lambda i: (0, 0)),
            pl.BlockSpec((1, 128), lambda i: (0, 0)),
        ],
        out_specs=pl.BlockSpec((blk, 128), lambda i: (i, 0)),
        out_shape=jax.ShapeDtypeStruct((N, 128), jnp.float32),
    )(qsum, dinv, ft, w1, wv, wf, b)


def _cnt_col(hist):
    return hist.reshape(80 * 128)[:N].reshape(N, 1)


def kernel(ft_a, ft_p, edge_a2p, edge_p2a, W0_ap, W0_pa, W1_ap, W1_pa,
           Wfc_a, bfc_a, Wfc_p, bfc_p):
    src_a, dst_a = _pack_edges(edge_a2p)
    src_p, dst_p = _pack_edges(edge_p2a)
    zeros = jnp.zeros((ZROWS, 128), jnp.float32)
    agg = _make_agg()

    cnt_a = _cnt_col(_tc_count(dst_a.reshape(HGRID, 1, HB)))
    cnt_p = _cnt_col(_tc_count(dst_p.reshape(HGRID, 1, HB)))

    sum_a1, sum_p1 = agg(ft_p, ft_a, src_a, dst_a, src_p, dst_p, zeros)
    emb1_a, dinv_a = _tc_layer1(sum_a1, cnt_a, W0_ap)
    emb1_p, dinv_p = _tc_layer1(sum_p1, cnt_p, W0_pa)

    sum_a2, sum_p2 = agg(emb1_p, emb1_a, src_a, dst_a, src_p, dst_p, zeros)
    out_a = _tc_layer2(sum_a2, dinv_a, ft_a, W1_ap,
                       Wfc_a[:128], Wfc_a[128:], bfc_a.reshape(1, 128))
    out_p = _tc_layer2(sum_p2, dinv_p, ft_p, W1_pa,
                       Wfc_p[:128], Wfc_p[128:], bfc_p.reshape(1, 128))
    return jnp.concatenate([out_a, out_p], axis=0)
